# Initial kernel scaffold; baseline (speedup 1.0000x reference)
import jax
import jax.numpy as jnp
from jax.experimental import pallas as pl


def kernel(x, codebook):
    raise NotImplementedError("write your pallas kernel here")


# speedup vs baseline: 1.0667x; 1.0667x over previous
import functools

import jax
import jax.numpy as jnp
from jax.experimental import pallas as pl


NUM_EMB = 1024
EMB_DIM = 64


def _vq_kernel(x_ref, cb_ref, q_bnc_ref, q_bcn_ref, loss_ref):
    b = pl.program_id(0)
    xc = x_ref[0]
    xp = xc.T
    cb = cb_ref[...]

    x_sq = jnp.sum(xp * xp, axis=-1, keepdims=True)
    e_sq = jnp.sum(cb * cb, axis=-1)
    dot = jnp.dot(xp, cb.T, preferred_element_type=jnp.float32)
    d2 = (x_sq + e_sq[None, :]) - 2.0 * dot
    dis = jnp.sqrt(jnp.maximum(d2, 0.0))
    m = jnp.min(dis, axis=-1, keepdims=True)
    iota_k = jax.lax.broadcasted_iota(jnp.int32, (1, NUM_EMB), 1)
    cand = jnp.where(dis == m, iota_k, NUM_EMB)
    z = jnp.min(cand, axis=-1)

    onehot = (z[:, None] == iota_k).astype(jnp.float32)
    q = jax.lax.dot_general(
        onehot, cb, (((1,), (0,)), ((), ())),
        precision=jax.lax.Precision.HIGHEST,
        preferred_element_type=jnp.float32)

    q_bnc_ref[0] = q
    qq = xp + (q - xp)
    q_bcn_ref[0] = qq.T

    diff = xp - q
    psum = jnp.sum(diff * diff).reshape(1, 1)

    @pl.when(b == 0)
    def _():
        loss_ref[...] = jnp.zeros((1, 1), jnp.float32)
    loss_ref[...] += psum


@functools.partial(jax.jit, static_argnames=())
def kernel(x, codebook):
    B, C, H, W = x.shape
    N = H * W
    x3 = x.reshape(B, C, N)

    q_bnc, q_bcn, loss_sum = pl.pallas_call(
        _vq_kernel,
        grid=(B,),
        in_specs=[
            pl.BlockSpec((1, C, N), lambda b: (b, 0, 0)),
            pl.BlockSpec((NUM_EMB, EMB_DIM), lambda b: (0, 0)),
        ],
        out_specs=[
            pl.BlockSpec((1, N, C), lambda b: (b, 0, 0)),
            pl.BlockSpec((1, C, N), lambda b: (b, 0, 0)),
            pl.BlockSpec((1, 1), lambda b: (0, 0)),
        ],
        out_shape=[
            jax.ShapeDtypeStruct((B, N, C), jnp.float32),
            jax.ShapeDtypeStruct((B, C, N), jnp.float32),
            jax.ShapeDtypeStruct((1, 1), jnp.float32),
        ],
    )(x3, codebook)

    n_elems = jnp.float32(B * N * C)
    commitment_loss = (loss_sum[0, 0] / n_elems).astype(jnp.float32)
    codebook_loss = commitment_loss
    quantizer_loss = jnp.float32(0.2) * commitment_loss + codebook_loss

    quantized = q_bcn.reshape(B, C, H, W)
    min_index_r = q_bnc.reshape(B, C, H, W)
    return (quantized, codebook_loss, commitment_loss, quantizer_loss, min_index_r)


# speedup vs baseline: 1.4864x; 1.3935x over previous
import functools

import jax
import jax.numpy as jnp
from jax.experimental import pallas as pl


NUM_EMB = 1024
EMB_DIM = 64


def _vq_kernel(x_ref, cb_ref, q_bnc_ref, q_bcn_ref, loss_ref):
    b = pl.program_id(0)
    xc = x_ref[0]
    xp = xc.T
    cb = cb_ref[...]

    x_sq = jnp.sum(xp * xp, axis=-1, keepdims=True)
    e_sq = jnp.sum(cb * cb, axis=-1)
    dot2 = jnp.dot(xp + xp, cb.T, preferred_element_type=jnp.float32)
    d2 = (x_sq + e_sq[None, :]) - dot2
    dis = jnp.sqrt(jnp.maximum(d2, 0.0))
    m = jnp.min(dis, axis=-1, keepdims=True)
    iota_k = jax.lax.broadcasted_iota(jnp.int32, (1, NUM_EMB), 1)
    cand = jnp.where(dis == m, iota_k, NUM_EMB)
    z = jnp.min(cand, axis=-1)

    onehot = (z[:, None] == iota_k).astype(jnp.float32)
    q = jnp.dot(onehot, cb, preferred_element_type=jnp.float32)

    q_bnc_ref[0] = q
    qq = xp + (q - xp)
    q_bcn_ref[0] = qq.T

    diff = xp - q
    psum = jnp.sum(diff * diff).reshape(1, 1)

    @pl.when(b == 0)
    def _():
        loss_ref[...] = jnp.zeros((1, 1), jnp.float32)
    loss_ref[...] += psum


@functools.partial(jax.jit, static_argnames=())
def kernel(x, codebook):
    B, C, H, W = x.shape
    N = H * W
    x3 = x.reshape(B, C, N)

    q_bnc, q_bcn, loss_sum = pl.pallas_call(
        _vq_kernel,
        grid=(B,),
        in_specs=[
            pl.BlockSpec((1, C, N), lambda b: (b, 0, 0)),
            pl.BlockSpec((NUM_EMB, EMB_DIM), lambda b: (0, 0)),
        ],
        out_specs=[
            pl.BlockSpec((1, N, C), lambda b: (b, 0, 0)),
            pl.BlockSpec((1, C, N), lambda b: (b, 0, 0)),
            pl.BlockSpec((1, 1), lambda b: (0, 0)),
        ],
        out_shape=[
            jax.ShapeDtypeStruct((B, N, C), jnp.float32),
            jax.ShapeDtypeStruct((B, C, N), jnp.float32),
            jax.ShapeDtypeStruct((1, 1), jnp.float32),
        ],
    )(x3, codebook)

    n_elems = jnp.float32(B * N * C)
    commitment_loss = (loss_sum[0, 0] / n_elems).astype(jnp.float32)
    codebook_loss = commitment_loss
    quantizer_loss = jnp.float32(0.2) * commitment_loss + codebook_loss

    quantized = q_bcn.reshape(B, C, H, W)
    min_index_r = q_bnc.reshape(B, C, H, W)
    return (quantized, codebook_loss, commitment_loss, quantizer_loss, min_index_r)


# speedup vs baseline: 1.6101x; 1.0832x over previous
import functools

import jax
import jax.numpy as jnp
from jax.experimental import pallas as pl


NUM_EMB = 1024
EMB_DIM = 64


def _vq_kernel(nbatch, x_ref, cb_ref, q_bnc_ref, q_bcn_ref, loss_ref):
    cb = cb_ref[...]
    e_sq = jnp.sum(cb * cb, axis=-1)
    iota_k = jax.lax.broadcasted_iota(jnp.int32, (1, NUM_EMB), 1)
    loss = jnp.zeros((1, 1), jnp.float32)
    for b in range(nbatch):
        xc = x_ref[b]
        xp = xc.T
        x_sq = jnp.sum(xp * xp, axis=-1, keepdims=True)
        dot2 = jnp.dot(xp + xp, cb.T, preferred_element_type=jnp.float32)
        d2 = (x_sq + e_sq[None, :]) - dot2
        dis = jnp.sqrt(jnp.maximum(d2, 0.0))
        m = jnp.min(dis, axis=-1, keepdims=True)
        cand = jnp.where(dis == m, iota_k, NUM_EMB)
        z = jnp.min(cand, axis=-1)
        onehot = (z[:, None] == iota_k).astype(jnp.float32)
        q = jnp.dot(onehot, cb, preferred_element_type=jnp.float32)
        q_bnc_ref[b] = q
        qq = xp + (q - xp)
        q_bcn_ref[b] = qq.T
        diff = xp - q
        loss = loss + jnp.sum(diff * diff).reshape(1, 1)
    loss_ref[...] = loss


@functools.partial(jax.jit, static_argnames=())
def kernel(x, codebook):
    B, C, H, W = x.shape
    N = H * W
    x3 = x.reshape(B, C, N)

    q_bnc, q_bcn, loss_sum = pl.pallas_call(
        functools.partial(_vq_kernel, B),
        out_shape=[
            jax.ShapeDtypeStruct((B, N, C), jnp.float32),
            jax.ShapeDtypeStruct((B, C, N), jnp.float32),
            jax.ShapeDtypeStruct((1, 1), jnp.float32),
        ],
    )(x3, codebook)

    n_elems = jnp.float32(B * N * C)
    commitment_loss = (loss_sum[0, 0] / n_elems).astype(jnp.float32)
    codebook_loss = commitment_loss
    quantizer_loss = jnp.float32(0.2) * commitment_loss + codebook_loss

    quantized = q_bcn.reshape(B, C, H, W)
    min_index_r = q_bnc.reshape(B, C, H, W)
    return (quantized, codebook_loss, commitment_loss, quantizer_loss, min_index_r)


# speedup vs baseline: 1.6251x; 1.0093x over previous
import functools

import jax
import jax.numpy as jnp
from jax.experimental import pallas as pl


NUM_EMB = 1024
EMB_DIM = 64
BATCH_PER_STEP = 4


def _vq_kernel(nbatch, x_ref, cb_ref, q_bnc_ref, q_bcn_ref, loss_ref):
    step = pl.program_id(0)
    cb = cb_ref[...]
    e_sq = jnp.sum(cb * cb, axis=-1)
    iota_k = jax.lax.broadcasted_iota(jnp.int32, (1, NUM_EMB), 1)
    loss = jnp.zeros((1, 1), jnp.float32)
    for b in range(nbatch):
        xc = x_ref[b]
        xp = xc.T
        x_sq = jnp.sum(xp * xp, axis=-1, keepdims=True)
        dot2 = jnp.dot(xp + xp, cb.T, preferred_element_type=jnp.float32)
        d2 = (x_sq + e_sq[None, :]) - dot2
        dis = jnp.sqrt(jnp.maximum(d2, 0.0))
        m = jnp.min(dis, axis=-1, keepdims=True)
        cand = jnp.where(dis == m, iota_k, NUM_EMB)
        z = jnp.min(cand, axis=-1)
        onehot = (z[:, None] == iota_k).astype(jnp.float32)
        q = jnp.dot(onehot, cb, preferred_element_type=jnp.float32)
        q_bnc_ref[b] = q
        qq = xp + (q - xp)
        q_bcn_ref[b] = qq.T
        diff = xp - q
        loss = loss + jnp.sum(diff * diff).reshape(1, 1)

    @pl.when(step == 0)
    def _():
        loss_ref[...] = jnp.zeros((1, 1), jnp.float32)
    loss_ref[...] += loss


@functools.partial(jax.jit, static_argnames=())
def kernel(x, codebook):
    B, C, H, W = x.shape
    N = H * W
    x3 = x.reshape(B, C, N)
    bps = BATCH_PER_STEP
    nsteps = B // bps

    q_bnc, q_bcn, loss_sum = pl.pallas_call(
        functools.partial(_vq_kernel, bps),
        grid=(nsteps,),
        in_specs=[
            pl.BlockSpec((bps, C, N), lambda i: (i, 0, 0)),
            pl.BlockSpec((NUM_EMB, EMB_DIM), lambda i: (0, 0)),
        ],
        out_specs=[
            pl.BlockSpec((bps, N, C), lambda i: (i, 0, 0)),
            pl.BlockSpec((bps, C, N), lambda i: (i, 0, 0)),
            pl.BlockSpec((1, 1), lambda i: (0, 0)),
        ],
        out_shape=[
            jax.ShapeDtypeStruct((B, N, C), jnp.float32),
            jax.ShapeDtypeStruct((B, C, N), jnp.float32),
            jax.ShapeDtypeStruct((1, 1), jnp.float32),
        ],
    )(x3, codebook)

    n_elems = jnp.float32(B * N * C)
    commitment_loss = (loss_sum[0, 0] / n_elems).astype(jnp.float32)
    codebook_loss = commitment_loss
    quantizer_loss = jnp.float32(0.2) * commitment_loss + codebook_loss

    quantized = q_bcn.reshape(B, C, H, W)
    min_index_r = q_bnc.reshape(B, C, H, W)
    return (quantized, codebook_loss, commitment_loss, quantizer_loss, min_index_r)
